# Initial kernel scaffold; baseline (speedup 1.0000x reference)
import jax
import jax.numpy as jnp
from jax.experimental import pallas as pl


def kernel(x, x0, token_ids, W_in, W_sel_in, W_sel_out, W_out, d_param, resid_mix, ssm_scale, mlp_scale, W_mlp1, W_mlp2):
    raise NotImplementedError("write your pallas kernel here")


# speedup vs baseline: 169.5571x; 169.5571x over previous
import functools

import jax
import jax.numpy as jnp
from jax.experimental import pallas as pl
from jax.experimental.pallas import tpu as pltpu

T = 2048
D = 768
E = 8
S = 128
H = 128
CHUNK = 256
NCHUNK = T // CHUNK


def _routes_from_tokens(tid):
    x = tid
    x = x ^ jax.lax.shift_right_logical(x, 16)
    x = x * jnp.int32(-2048144789)
    x = x ^ jax.lax.shift_right_logical(x, 13)
    x = x * jnp.int32(-1028477387)
    x = x ^ jax.lax.shift_right_logical(x, 16)
    return x & jnp.int32(E - 1)


def _gates_kernel(x_ref, x0_ref, tid_ref, win_ref, wsi_ref, wso_ref, dp_ref,
                  rm_ref, xm_ref, r_ref, a_ref, bu_ref, c_ref, dd_ref):
    rm = rm_ref[...]
    xm = rm[0:1, :] * x_ref[...] + rm[1:2, :] * x0_ref[...]
    xm_ref[...] = xm
    ms = jnp.mean(xm * xm, axis=1, keepdims=True)
    xn = xm * jax.lax.rsqrt(ms + 1e-6)

    r = _routes_from_tokens(tid_ref[...])
    r_ref[...] = r

    f32 = jnp.float32
    u = jnp.zeros((CHUNK, S), f32)
    selz = jnp.zeros((CHUNK, H), f32)
    for e in range(E):
        m = (r == e).astype(f32)
        xe = xn * m
        u = u + jnp.dot(xe, win_ref[e], preferred_element_type=f32)
        selz = selz + jnp.dot(xe, wsi_ref[e], preferred_element_type=f32)
    sel = selz * jax.nn.sigmoid(selz)
    so = jnp.zeros((CHUNK, 4 * S), f32)
    dp = jnp.zeros((CHUNK, S), f32)
    for e in range(E):
        m = (r == e).astype(f32)
        so = so + jnp.dot(sel * m, wso_ref[e], preferred_element_type=f32)
        dp = dp + m * dp_ref[e:e + 1, :]
    a = jax.nn.sigmoid(so[:, 0:S])
    b = jnp.tanh(so[:, S:2 * S])
    c = jnp.tanh(so[:, 2 * S:3 * S])
    dg = jax.nn.sigmoid(so[:, 3 * S:4 * S])
    a_ref[...] = a
    bu_ref[...] = b * u
    c_ref[...] = c
    dd_ref[...] = dp * dg * u


def _scan_kernel(r_ref, a_ref, bu_ref, c_ref, dd_ref, y_ref):
    eidx = jax.lax.broadcasted_iota(jnp.int32, (E, 1), 0)

    def body(t, h):
        rt = r_ref[t]
        at = a_ref[pl.ds(t, 1), :]
        but = bu_ref[pl.ds(t, 1), :]
        ct = c_ref[pl.ds(t, 1), :]
        ddt = dd_ref[pl.ds(t, 1), :]
        mask = eidx == rt
        hn = jnp.where(mask, at * h + but, h)
        hr = jnp.sum(jnp.where(mask, hn, 0.0), axis=0, keepdims=True)
        y_ref[pl.ds(t, 1), :] = ct * hr + ddt
        return hn

    jax.lax.fori_loop(0, T, body, jnp.zeros((E, S), jnp.float32))


def _out_kernel(y_ref, r_ref, xm_ref, wout_ref, ssm_ref, mlp_ref,
                w1_ref, w2_ref, o_ref):
    f32 = jnp.float32
    r = r_ref[...]
    y = y_ref[...]
    out = jnp.zeros((CHUNK, D), f32)
    for e in range(E):
        m = (r == e).astype(f32)
        out = out + jnp.dot(y * m, wout_ref[e], preferred_element_type=f32)
    xm2 = xm_ref[...] + ssm_ref[...] * out
    ms = jnp.mean(xm2 * xm2, axis=1, keepdims=True)
    xn2 = xm2 * jax.lax.rsqrt(ms + 1e-6)
    hmid = jnp.dot(xn2, w1_ref[...], preferred_element_type=f32)
    hmid = jnp.square(jnp.maximum(hmid, 0.0))
    mlp = jnp.dot(hmid, w2_ref[...], preferred_element_type=f32)
    o_ref[...] = xm2 + mlp_ref[...] * mlp


def kernel(x, x0, token_ids, W_in, W_sel_in, W_sel_out, W_out, d_param,
           resid_mix, ssm_scale, mlp_scale, W_mlp1, W_mlp2):
    f32 = jnp.float32
    x2 = x.reshape(T, D)
    x02 = x0.reshape(T, D)
    tid = token_ids.reshape(T, 1)

    full = lambda shape: pl.BlockSpec(shape, lambda i: tuple(0 for _ in shape))
    chunk = lambda shape: pl.BlockSpec(shape, lambda i: (i,) + tuple(0 for _ in shape[1:]))

    xm, r, a, bu, c, dd = pl.pallas_call(
        _gates_kernel,
        grid=(NCHUNK,),
        in_specs=[
            chunk((CHUNK, D)), chunk((CHUNK, D)), chunk((CHUNK, 1)),
            full((E, D, S)), full((E, D, H)), full((E, H, 4 * S)),
            full((E, S)), full((2, D)),
        ],
        out_specs=[
            chunk((CHUNK, D)), chunk((CHUNK, 1)), chunk((CHUNK, S)),
            chunk((CHUNK, S)), chunk((CHUNK, S)), chunk((CHUNK, S)),
        ],
        out_shape=[
            jax.ShapeDtypeStruct((T, D), f32),
            jax.ShapeDtypeStruct((T, 1), jnp.int32),
            jax.ShapeDtypeStruct((T, S), f32),
            jax.ShapeDtypeStruct((T, S), f32),
            jax.ShapeDtypeStruct((T, S), f32),
            jax.ShapeDtypeStruct((T, S), f32),
        ],
    )(x2, x02, tid, W_in, W_sel_in, W_sel_out, d_param, resid_mix)

    y = pl.pallas_call(
        _scan_kernel,
        grid_spec=pltpu.PrefetchScalarGridSpec(
            num_scalar_prefetch=1,
            grid=(1,),
            in_specs=[
                pl.BlockSpec((T, S), lambda i, s: (0, 0)),
                pl.BlockSpec((T, S), lambda i, s: (0, 0)),
                pl.BlockSpec((T, S), lambda i, s: (0, 0)),
                pl.BlockSpec((T, S), lambda i, s: (0, 0)),
            ],
            out_specs=pl.BlockSpec((T, S), lambda i, s: (0, 0)),
        ),
        out_shape=jax.ShapeDtypeStruct((T, S), f32),
    )(r.reshape(T), a, bu, c, dd)

    o = pl.pallas_call(
        _out_kernel,
        grid=(NCHUNK,),
        in_specs=[
            chunk((CHUNK, S)), chunk((CHUNK, 1)), chunk((CHUNK, D)),
            full((E, S, D)), full((1, D)), full((1, D)),
            full((D, 4 * D)), full((4 * D, D)),
        ],
        out_specs=chunk((CHUNK, D)),
        out_shape=jax.ShapeDtypeStruct((T, D), f32),
    )(y, r, xm, W_out, ssm_scale.reshape(1, D), mlp_scale.reshape(1, D),
      W_mlp1, W_mlp2)

    return o.reshape(1, T, D)
